# baseline (speedup 1.0000x reference)
import functools

import jax
import jax.numpy as jnp
from jax import lax
from jax.experimental import pallas as pl
from jax.experimental.pallas import tpu as pltpu
from jax.experimental.pallas import tpu_sc as plsc

_NC = 2
_NS = 16
_NW = _NC * _NS
_L = 16

_CHUNK_ROWS = 8


def _make_sc_add(batch, seq, d):
    rows_per_w = seq // _NW
    n_chunks = rows_per_w // _CHUNK_ROWS
    chunk = _CHUNK_ROWS * d

    mesh = plsc.VectorSubcoreMesh(core_axis_name="c", subcore_axis_name="s")

    nring = 3
    vmem = []
    for _ in range(nring):
        vmem.append(pltpu.VMEM((chunk,), jnp.float32))
        vmem.append(pltpu.VMEM((batch, chunk), jnp.float32))
    sems = [pltpu.SemaphoreType.DMA for _ in range(2 * nring)]

    @functools.partial(
        pl.kernel,
        mesh=mesh,
        out_type=jax.ShapeDtypeStruct((batch, seq * d), jnp.float32),
        scratch_types=vmem + sems,
    )
    def sc_add(x_hbm, pos_hbm, out_hbm, *scratch):
        pos_v = tuple(scratch[2 * i] for i in range(nring))
        x_v = tuple(scratch[2 * i + 1] for i in range(nring))
        sem_l = scratch[2 * nring:]
        in_sem = tuple(sem_l[:nring])
        out_sem = tuple(sem_l[nring:])

        wid = lax.axis_index("s") * _NC + lax.axis_index("c")
        base = wid * rows_per_w * d

        def start_in(c, slot):
            off = base + c * chunk
            copies = [
                pltpu.async_copy(pos_hbm.at[pl.ds(off, chunk)],
                                 pos_v[slot], in_sem[slot]),
            ]
            for b in range(batch):
                copies.append(
                    pltpu.async_copy(x_hbm.at[b, pl.ds(off, chunk)],
                                     x_v[slot].at[b], in_sem[slot]))
            return copies

        in_handles = [None] * nring
        out_handles = [None] * nring
        in_handles[0] = start_in(0, 0)
        for c in range(n_chunks):
            slot = c % nring
            if c + 1 < n_chunks:
                nxt = (c + 1) % nring
                if out_handles[nxt] is not None:
                    for h in out_handles[nxt]:
                        h.wait()
                    out_handles[nxt] = None
                in_handles[nxt] = start_in(c + 1, nxt)
            for h in in_handles[slot]:
                h.wait()

            @pl.loop(0, chunk, step=_L)
            def _body(r, slot=slot):
                sl = pl.ds(r, _L)
                p = pos_v[slot][sl]
                for b in range(batch):
                    x_v[slot][b, sl] = x_v[slot][b, sl] + p

            off = base + c * chunk
            out_handles[slot] = [
                pltpu.async_copy(x_v[slot].at[b],
                                 out_hbm.at[b, pl.ds(off, chunk)],
                                 out_sem[slot])
                for b in range(batch)
            ]
        for hs in out_handles:
            if hs is not None:
                for h in hs:
                    h.wait()

    return sc_add


def kernel(x, pos_table):
    batch, seq, d = x.shape
    pos = pos_table[:seq]
    out = _make_sc_add(batch, seq, d)(
        x.reshape(batch, seq * d), pos.reshape(seq * d))
    return out.reshape(batch, seq, d)


# speedup vs baseline: 1.1794x; 1.1794x over previous
import functools

import jax
import jax.numpy as jnp
from jax import lax
from jax.experimental import pallas as pl
from jax.experimental.pallas import tpu as pltpu
from jax.experimental.pallas import tpu_sc as plsc

_NC = 2
_NS = 16
_NW = _NC * _NS
_L = 16

_CHUNK_ROWS = 8


def _make_sc_add(batch, seq, d):
    rows_per_w = seq // _NW
    n_chunks = rows_per_w // _CHUNK_ROWS
    chunk = _CHUNK_ROWS * d

    mesh = plsc.VectorSubcoreMesh(core_axis_name="c", subcore_axis_name="s")

    nring = 3
    vmem = []
    for _ in range(nring):
        vmem.append(pltpu.VMEM((chunk,), jnp.float32))
        for _ in range(batch):
            vmem.append(pltpu.VMEM((chunk,), jnp.float32))
    sems = [pltpu.SemaphoreType.DMA for _ in range(2 * nring)]

    @functools.partial(
        pl.kernel,
        mesh=mesh,
        out_type=jax.ShapeDtypeStruct((batch, seq * d), jnp.float32),
        scratch_types=vmem + sems,
    )
    def sc_add(x_hbm, pos_hbm, out_hbm, *scratch):
        stride = 1 + batch
        pos_v = tuple(scratch[stride * i] for i in range(nring))
        x_v = tuple(scratch[stride * i + 1:stride * (i + 1)]
                    for i in range(nring))
        sem_l = scratch[stride * nring:]
        in_sem = tuple(sem_l[:nring])
        out_sem = tuple(sem_l[nring:])

        wid = lax.axis_index("s") * _NC + lax.axis_index("c")
        base = wid * rows_per_w * d

        def start_in(c, slot):
            off = base + c * chunk
            copies = [
                pltpu.async_copy(pos_hbm.at[pl.ds(off, chunk)],
                                 pos_v[slot], in_sem[slot]),
            ]
            for b in range(batch):
                copies.append(
                    pltpu.async_copy(x_hbm.at[b, pl.ds(off, chunk)],
                                     x_v[slot][b], in_sem[slot]))
            return copies

        in_handles = [None] * nring
        out_handles = [None] * nring
        in_handles[0] = start_in(0, 0)
        for c in range(n_chunks):
            slot = c % nring
            if c + 1 < n_chunks:
                nxt = (c + 1) % nring
                if out_handles[nxt] is not None:
                    for h in out_handles[nxt]:
                        h.wait()
                    out_handles[nxt] = None
                in_handles[nxt] = start_in(c + 1, nxt)
            for h in in_handles[slot]:
                h.wait()

            @pl.loop(0, chunk, step=_L)
            def _body(r, slot=slot):
                sl = pl.ds(r, _L)
                p = pos_v[slot][sl]
                for b in range(batch):
                    x_v[slot][b][sl] = x_v[slot][b][sl] + p

            off = base + c * chunk
            out_handles[slot] = [
                pltpu.async_copy(x_v[slot][b],
                                 out_hbm.at[b, pl.ds(off, chunk)],
                                 out_sem[slot])
                for b in range(batch)
            ]
        for hs in out_handles:
            if hs is not None:
                for h in hs:
                    h.wait()

    return sc_add


def kernel(x, pos_table):
    batch, seq, d = x.shape
    pos = pos_table[:seq]
    out = _make_sc_add(batch, seq, d)(
        x.reshape(batch, seq * d), pos.reshape(seq * d))
    return out.reshape(batch, seq, d)
